# baseline (speedup 1.0000x reference)
import functools

import jax
import jax.numpy as jnp
from jax import lax
from jax.experimental import pallas as pl
from jax.experimental.pallas import tpu as pltpu
from jax.experimental.pallas import tpu_sc as plsc

NUM_GRAPHS = 256
D = 128
CH = 128
L = 16
NV = D // L
NBUF = 3

_info = plsc.get_sparse_core_info()
NC = _info.num_cores
NS = _info.num_subcores
NW = NC * NS

N_CELLS = 320000
N_TRACKS = 160000
CELL_CHUNKS = N_CELLS // CH
TRACK_CHUNKS = N_TRACKS // CH
NKMAX_C = 80
NKMAX_T = 40


def _sc_partials(cells_feat, cells_ids, tracks_feat, tracks_ids):
    mesh = plsc.VectorSubcoreMesh(core_axis_name="c", subcore_axis_name="s")
    f32 = jnp.float32
    i32 = jnp.int32

    @functools.partial(
        pl.kernel,
        mesh=mesh,
        compiler_params=pltpu.CompilerParams(needs_layout_passes=False),
        out_type=[
            jax.ShapeDtypeStruct((NW, NUM_GRAPHS, D), f32),
            jax.ShapeDtypeStruct((NW, NUM_GRAPHS, D), f32),
            jax.ShapeDtypeStruct((NC, NUM_GRAPHS, D), f32),
            jax.ShapeDtypeStruct((NC, NUM_GRAPHS, D), f32),
            jax.ShapeDtypeStruct((NW, NUM_GRAPHS, L), i32),
            jax.ShapeDtypeStruct((NW, NUM_GRAPHS, L), i32),
        ],
        scratch_types=[
            pltpu.VMEM((CH, D), f32),
            pltpu.VMEM((CH, D), f32),
            pltpu.VMEM((CH, D), f32),
            pltpu.VMEM((NKMAX_C, CH), i32),
            pltpu.VMEM((NUM_GRAPHS, D), f32),
            pltpu.VMEM((NUM_GRAPHS, L), i32),
            pltpu.VMEM_SHARED((NUM_GRAPHS, D), f32),
            pltpu.SemaphoreType.DMA,
            pltpu.SemaphoreType.DMA,
            pltpu.SemaphoreType.DMA,
            pltpu.SemaphoreType.DMA,
        ],
    )
    def k(cells_hbm, cids_hbm, tracks_hbm, tids_hbm,
          out_cs, out_ts, out_cs2, out_ts2, out_pc, out_pt,
          rows0_v, rows1_v, rows2_v, ids_v, acc_l, pos_l, acc_s,
          sem_r0, sem_r1, sem_r2, sem_i):
        c = lax.axis_index("c")
        s = lax.axis_index("s")
        w = s * NC + c

        zero16 = jnp.zeros((L,), f32)
        izero16 = jnp.zeros((L,), i32)

        def zero_acc():
            def zb(i, _):
                for jj in range(NV):
                    acc_l[i, pl.ds(jj * L, L)] = zero16
                pos_l[i, pl.ds(0, L)] = izero16
                return 0
            lax.fori_loop(0, NUM_GRAPHS, zb, 0)

        zero_acc()

        lane = lax.iota(i32, L)
        zoff = w * 0
        lanes_j = [lane + (zoff + jj * L) for jj in range(NV)]
        zeros_idx = jnp.zeros((L,), i32) + zoff
        _dn = lax.GatherDimensionNumbers(
            offset_dims=(), collapsed_slice_dims=(0,), start_index_map=(0,))

        def take16(vec, idx):
            return lax.gather(vec, idx[:, None], _dn, (1,),
                              mode=lax.GatherScatterMode.PROMISE_IN_BOUNDS)

        rbufs = ((rows0_v, sem_r0), (rows1_v, sem_r1), (rows2_v, sem_r2))
        sl = pl.ds(s * (NUM_GRAPHS // NS), NUM_GRAPHS // NS)

        def run_array(feat_hbm, ids_2d, nk, st, nk_max):
            pltpu.sync_copy(ids_2d.at[pl.ds(st, nk_max)],
                            ids_v.at[pl.ds(0, nk_max)], )

            def start_load(gg, rows_b, sr):
                base = pl.multiple_of(gg * CH, CH)
                pltpu.async_copy(feat_hbm.at[pl.ds(base, CH)], rows_b, sr)

            def wait_load(rows_b, sr):
                pltpu.make_async_copy(feat_hbm.at[pl.ds(0, CH)], rows_b,
                                      sr).wait()

            def issue_stream(g, rows_b):
                pltpu.async_copy(rows_b, acc_s.at[ids_v.at[g]], sem_i,
                                 add=True)

            def drain_stream(g, rows_b):
                pltpu.make_async_copy(rows_b, acc_s.at[ids_v.at[g]],
                                      sem_i).wait()

            def pos_scatter(g, gg):
                rowbase = gg * CH + 1

                def pbody(v, carry):
                    idv = ids_v[g, pl.ds(v * L, L)]
                    posv = (rowbase + v * L) + lane
                    plsc.store_scatter(pos_l, [idv, lane], posv)
                    return carry
                lax.fori_loop(0, CH // L, pbody, 0)

            def process(g, rows_b):
                def gbody(v, carry):
                    idv = ids_v[g, pl.ds(v * L, L)]
                    gfirst = take16(idv, zeros_idx)
                    uni = jnp.min(jnp.where(idv == gfirst, 1, 0))
                    base_r = v * L

                    @pl.when(uni == 1)
                    def _():
                        accs = [rows_b[base_r, pl.ds(jj * L, L)]
                                for jj in range(NV)]
                        for r in range(1, L):
                            for jj in range(NV):
                                accs[jj] = accs[jj] + rows_b[
                                    base_r + r, pl.ds(jj * L, L)]
                        for jj in range(NV):
                            plsc.addupdate_scatter(
                                acc_l, [gfirst, lanes_j[jj]], accs[jj])

                    @pl.when(uni == 0)
                    def _():
                        def rbody(r, c2):
                            gid = take16(idv, zeros_idx + r)
                            for jj in range(NV):
                                plsc.addupdate_scatter(
                                    acc_l, [gid, lanes_j[jj]],
                                    rows_b[base_r + r, pl.ds(jj * L, L)])
                            return c2
                        lax.fori_loop(0, L, rbody, 0)
                    return carry

                lax.fori_loop(0, CH // L, gbody, 0)

            start_load(st, *rbufs[0])

            @pl.when(nk > 1)
            def _():
                start_load(st + 1, *rbufs[1])

            def body(g, start):
                gg = start + g
                for r in range(NBUF):
                    @pl.when(g % NBUF == r)
                    def _():
                        if r in (1, 2):
                            drain_stream(g - 1, rbufs[r - 1][0])

                        @pl.when(g + 2 < nk)
                        def _():
                            start_load(gg + 2, *rbufs[(r + 2) % NBUF])
                        wait_load(*rbufs[r])
                        pos_scatter(g, gg)
                        if r in (0, 1):
                            issue_stream(g, rbufs[r][0])
                        else:
                            process(g, rbufs[r][0])
                return start

            lax.fori_loop(0, nk, body, st)

            for r in (0, 1):
                @pl.when((nk - 1) % NBUF == r)
                def _():
                    drain_stream(nk - 1, rbufs[r][0])

        def phase(feat_hbm, ids_2d, nk, st, nk_max, out_tec, out_stream,
                  out_pos):
            pltpu.sync_copy(acc_l.at[sl], acc_s.at[sl])
            plsc.subcore_barrier()
            run_array(feat_hbm, ids_2d, nk, st, nk_max)
            plsc.subcore_barrier()
            pltpu.sync_copy(acc_l, out_tec.at[w])
            pltpu.sync_copy(pos_l, out_pos.at[w])
            pltpu.sync_copy(acc_s.at[sl], out_stream.at[c, sl])
            plsc.subcore_barrier()
            zero_acc()

        nk_c = jnp.where(w < 24, 80, jnp.where(w < 31, 72, 76))
        st_c = jnp.where(w < 24, 80 * w, 1920 + 72 * (w - 24))
        phase(cells_hbm, cids_hbm, nk_c, st_c, NKMAX_C, out_cs, out_cs2,
              out_pc)

        nk_t = jnp.where(w < 28, 40, jnp.where(w < 31, 32, 34))
        st_t = jnp.where(w < 28, 40 * w, 1120 + 32 * (w - 28))
        phase(tracks_hbm, tids_hbm, nk_t, st_t, NKMAX_T, out_ts, out_ts2,
              out_pt)

    cids2 = jnp.pad(cells_ids.reshape(-1, CH), ((0, 8), (0, 0)))
    tids2 = jnp.pad(tracks_ids.reshape(-1, CH), ((0, 8), (0, 0)))
    return k(cells_feat, cids2, tracks_feat, tids2)


def _counts_from_endpos(p_ref):
    e = jnp.max(p_ref[...], axis=(0, 2))[:, None]
    m = e
    sh = 1
    while sh < NUM_GRAPHS:
        z = jnp.zeros((sh, 1), m.dtype)
        m = jnp.maximum(m, jnp.concatenate([z, m[:-sh]], axis=0))
        sh *= 2
    prev = jnp.concatenate([jnp.zeros((1, 1), m.dtype), m[:-1]], axis=0)
    return (m - prev).astype(jnp.float32)


def _tc_head_body(cs_ref, ts_ref, cs2_ref, ts2_ref, pc_ref, pt_ref,
                  w_ref, b_ref, o_ref):
    cs = jnp.sum(cs_ref[...], axis=0) + cs2_ref[0] + cs2_ref[1]
    ts = jnp.sum(ts_ref[...], axis=0) + ts2_ref[0] + ts2_ref[1]
    cc = _counts_from_endpos(pc_ref)
    tc = _counts_from_endpos(pt_ref)
    mc = cs / jnp.maximum(cc, 1.0)
    mt = ts / jnp.maximum(tc, 1.0)
    ag = jnp.concatenate([mc, mt], axis=1)
    o_ref[...] = (
        jnp.dot(ag, w_ref[...], preferred_element_type=jnp.float32) + b_ref[...]
    )


def _tc_head(cs, ts, cs2, ts2, pc, pt, W, b):
    return pl.pallas_call(
        _tc_head_body,
        out_shape=jax.ShapeDtypeStruct((NUM_GRAPHS, 2), jnp.float32),
    )(cs, ts, cs2, ts2, pc, pt, W, b.reshape(1, 2))


def kernel(cells_feat, tracks_feat, W, b, cells_segment_ids, tracks_segment_ids):
    cids = cells_segment_ids.astype(jnp.int32)
    tids = tracks_segment_ids.astype(jnp.int32)
    cs, ts, cs2, ts2, pc, pt = _sc_partials(cells_feat, cids,
                                            tracks_feat, tids)
    return _tc_head(cs, ts, cs2, ts2, pc, pt, W, b)
